# Initial kernel scaffold; baseline (speedup 1.0000x reference)
import jax
import jax.numpy as jnp
from jax.experimental import pallas as pl


def kernel(Xi_one, Xi_mul, Xi_mle, bias, W1_one, W1_mul, W2_one, W2_mul, Wl1, bl1, Wl2, bl2, bn1_g, bn1_b, bn2_g, bn2_b):
    raise NotImplementedError("write your pallas kernel here")


# speedup vs baseline: 4.3844x; 4.3844x over previous
import functools

import numpy as np
import jax
import jax.numpy as jnp
from jax import lax
from jax.experimental import pallas as pl
from jax.experimental.pallas import tpu as pltpu
from jax.experimental.pallas import tpu_sc as plsc

F = 26
NONE = 21
NMUL = 5
V1 = 100000
VM = 100001
EMB = 16
MAXN = 20
B = 4096
PAIRS = F * (F - 1) // 2
D1 = 128
D2 = 128
BN_EPS = 1e-5

NW = 32
BPW = B // NW


_my_pairs = []
for _i in range(NONE):
    for _j in range(_i + 1, NONE):
        _my_pairs.append((_i, _j))
for _i in range(NONE):
    for _m in range(NMUL):
        _my_pairs.append((_i, NONE + _m))
_t10 = {}
for _a in range(NMUL):
    for _b in range(_a + 1, NMUL):
        _t10[(_a, _b)] = len(_t10)
        _my_pairs.append((NONE + _a, NONE + _b))
assert len(_my_pairs) == PAIRS

_slot_view = np.zeros(2 * PAIRS, np.int64)
_slot_field = np.zeros(2 * PAIRS, np.int64)
for _p, (_i, _j) in enumerate(_my_pairs):
    _slot_view[_p], _slot_field[_p] = _i, _j
    _slot_view[PAIRS + _p], _slot_field[PAIRS + _p] = _j, _i

_cov = np.zeros((F, F), np.int64)
for _s in range(2 * PAIRS):
    _cov[_slot_view[_s], _slot_field[_s]] += 1
assert (np.diag(_cov) == 0).all() and ((_cov + np.eye(F, dtype=np.int64)) == 1).all()

_iu, _ju = np.triu_indices(F, 1)
_ref_q = {(int(a), int(b)): q for q, (a, b) in enumerate(zip(_iu, _ju))}
_pair_perm = np.array([_ref_q[p] for p in _my_pairs], np.int64)
_WL1_ROW_PERM = (_pair_perm[:, None] * EMB + np.arange(EMB)[None, :]).reshape(-1)

_groups = [(v, m) for v in range(NONE) for m in range(NMUL)]
for _a in range(NMUL):
    for _b in range(NMUL):
        if _a != _b:
            _groups.append((NONE + _a, _b))
assert len(_groups) == 125


def _mul_dest(v, m):
    j = NONE + m
    if v < j:
        if v < NONE:
            return 210 + v * NMUL + m
        return 315 + _t10[(v - NONE, m)]
    return PAIRS + 315 + _t10[(m, v - NONE)]


_GROUP_DEST = np.array([_mul_dest(v, m) for v, m in _groups], np.int64)
for _g in range(105):
    assert _GROUP_DEST[_g] == 210 + _g
_MM_GROUPS = [(g, _groups[g][1], int(_GROUP_DEST[g])) for g in range(105, 125)]
_sv = set(int(s) for s in _GROUP_DEST)
_mul_slots = set(int(s) for s in range(2 * PAIRS) if _slot_field[s] >= NONE)
assert _sv == _mul_slots

AB0, MR0, FO0, IDXL = 0, 656, 3160, 3360
_SRC = np.full(IDXL, 121, np.int64)
_OFF = np.zeros(IDXL, np.int64)
for _s in range(2 * PAIRS):
    if _slot_field[_s] < NONE:
        _SRC[_s] = _slot_field[_s]
        _OFF[_s] = _slot_view[_s] * V1
for _g, (_v, _m) in enumerate(_groups):
    for _k in range(MAXN):
        _e = MR0 + _g * MAXN + _k
        _SRC[_e] = NONE + _m * MAXN + _k
        _OFF[_e] = _v * VM
for _k in range(32):
    _e = FO0 + _k
    if _k < NONE:
        _SRC[_e] = _k
        _OFF[_e] = 0
    else:
        _OFF[_e] = V1
for _m in range(NMUL):
    for _k in range(32):
        _e = FO0 + 32 + _m * 32 + _k
        if _k < MAXN:
            _SRC[_e] = NONE + _m * MAXN + _k
        _OFF[_e] = V1
_SRC_J = _SRC.astype(np.int32)
_OFF_J = _OFF.astype(np.int32)

_AB_CH = [(0, 128), (128, 128), (256, 128), (384, 128), (512, 128), (640, 10)]
_MR_CH = [(j * 128, 128) for j in range(19)] + [(2432, 68)]
_FO_CH = [(0, 128), (128, 64)]


def _sc_gather(w2one, w2mul, w1cat, idx_all, len16):
    mesh = plsc.VectorSubcoreMesh(core_axis_name="c", subcore_axis_name="s")

    @functools.partial(
        pl.kernel,
        out_type=[
            jax.ShapeDtypeStruct((B, PAIRS, EMB), jnp.float32),
            jax.ShapeDtypeStruct((B, PAIRS, EMB), jnp.float32),
            jax.ShapeDtypeStruct((B, EMB), jnp.float32),
        ],
        mesh=mesh,
        scratch_types=[
            pltpu.VMEM((IDXL,), jnp.int32),
            pltpu.VMEM((2 * PAIRS, EMB), jnp.float32),
            pltpu.VMEM((2500, EMB), jnp.float32),
            pltpu.VMEM((192, EMB), jnp.float32),
            pltpu.VMEM((NMUL, EMB), jnp.float32),
            pltpu.VMEM((BPW, EMB), jnp.float32),
            pltpu.SemaphoreType.DMA,
        ],
        compiler_params=pltpu.CompilerParams(use_tc_tiling_on_sc=False),
    )
    def k(w2one_r, w2mul_r, w1cat_r, idx_r, len_r, a_out, b_out, fo_out,
          idx_v, ab_v, mr_v, fo_v, len_v, fov_v, sem):
        wid = lax.axis_index("c") * 16 + lax.axis_index("s")
        base = wid * BPW

        @pl.loop(0, BPW)
        def _row(e):
            bi = base + e
            pltpu.sync_copy(idx_r.at[bi], idx_v)
            pltpu.sync_copy(len_r.at[bi], len_v)
            cps = []
            for off, sz in _AB_CH:
                cps.append(pltpu.async_copy(
                    w2one_r.at[idx_v.at[pl.ds(AB0 + off, sz)]],
                    ab_v.at[pl.ds(off, sz)], sem))
            for off, sz in _MR_CH:
                cps.append(pltpu.async_copy(
                    w2mul_r.at[idx_v.at[pl.ds(MR0 + off, sz)]],
                    mr_v.at[pl.ds(off, sz)], sem))
            for off, sz in _FO_CH:
                cps.append(pltpu.async_copy(
                    w1cat_r.at[idx_v.at[pl.ds(FO0 + off, sz)]],
                    fo_v.at[pl.ds(off, sz)], sem))
            for c in cps:
                c.wait()

            @pl.loop(0, 105)
            def _grp(g):
                r0 = g * MAXN
                acc = mr_v[pl.ds(r0, 1), :]
                for kk in range(1, MAXN):
                    acc = acc + mr_v[pl.ds(r0 + kk, 1), :]
                m = lax.rem(g, NMUL)
                ab_v[pl.ds(210 + g, 1), :] = acc / len_v[pl.ds(m, 1), :]

            for g, m, dest in _MM_GROUPS:
                r0 = g * MAXN
                acc = mr_v[pl.ds(r0, 1), :]
                for kk in range(1, MAXN):
                    acc = acc + mr_v[pl.ds(r0 + kk, 1), :]
                ab_v[pl.ds(dest, 1), :] = acc / len_v[pl.ds(m, 1), :]

            facc = fo_v[pl.ds(0, 1), :]
            for kk in range(1, 32):
                facc = facc + fo_v[pl.ds(kk, 1), :]
            for m in range(NMUL):
                mac = fo_v[pl.ds(32 + m * 32, 1), :]
                for kk in range(1, 32):
                    mac = mac + fo_v[pl.ds(32 + m * 32 + kk, 1), :]
                facc = facc + mac / len_v[pl.ds(m, 1), :]
            fov_v[pl.ds(e, 1), :] = facc

            pltpu.sync_copy(ab_v.at[pl.ds(0, PAIRS)], a_out.at[bi])
            pltpu.sync_copy(ab_v.at[pl.ds(PAIRS, PAIRS)], b_out.at[bi])

        pltpu.sync_copy(fov_v, fo_out.at[pl.ds(base, BPW)])

    return k(w2one, w2mul, w1cat, idx_all, len16)


def _tc_body(a_ref, b_ref, fo_ref, w1_ref, c1_ref, w2_ref, c2_ref, bias_ref,
             o_ref):
    d = a_ref[...] * b_ref[...]
    x1 = jnp.dot(d, w1_ref[...], preferred_element_type=jnp.float32)
    x1 = jnp.maximum(x1 + c1_ref[...], 0.0)
    x2 = jnp.dot(x1, w2_ref[...], preferred_element_type=jnp.float32)
    x2 = jnp.maximum(x2 + c2_ref[...], 0.0)
    tot = bias_ref[0] + fo_ref[...].sum(axis=1) + d.sum(axis=1) + x2.sum(axis=1)
    o_ref[...] = tot


def _tc_mlp(a2, b2, fov, w1f, c1, w2f, c2, bias):
    blk = 128
    nblk = B // blk
    out = pl.pallas_call(
        _tc_body,
        grid=(nblk,),
        in_specs=[
            pl.BlockSpec((blk, PAIRS * EMB), lambda i: (i, 0)),
            pl.BlockSpec((blk, PAIRS * EMB), lambda i: (i, 0)),
            pl.BlockSpec((blk, EMB), lambda i: (i, 0)),
            pl.BlockSpec((PAIRS * EMB, D1), lambda i: (0, 0)),
            pl.BlockSpec((1, D1), lambda i: (0, 0)),
            pl.BlockSpec((D1, D2), lambda i: (0, 0)),
            pl.BlockSpec((1, D2), lambda i: (0, 0)),
            pl.BlockSpec(memory_space=pltpu.SMEM),
        ],
        out_specs=pl.BlockSpec((blk,), lambda i: (i,)),
        out_shape=jax.ShapeDtypeStruct((B,), jnp.float32),
    )(a2, b2, fov, w1f, c1, w2f, c2, bias)
    return out


def kernel(Xi_one, Xi_mul, Xi_mle, bias, W1_one, W1_mul, W2_one, W2_mul,
           Wl1, bl1, Wl2, bl2, bn1_g, bn1_b, bn2_g, bn2_b):
    Xi_one = Xi_one.astype(jnp.int32)
    Xi_mul = Xi_mul.astype(jnp.int32)
    Xi_mle = Xi_mle.astype(jnp.int32)

    w2one = W2_one.reshape(F * V1, EMB)
    w2mul = W2_mul.reshape(F * VM, EMB)
    w1flat = jnp.concatenate([W1_one[:, 0], W1_mul[:, 0]])
    w1cat = jnp.pad(w1flat[:, None], ((0, 0), (0, EMB - 1)))

    xcat = jnp.concatenate(
        [Xi_one, Xi_mul.reshape(B, NMUL * MAXN), jnp.zeros((B, 1), jnp.int32)],
        axis=1)
    idx_all = xcat[:, _SRC_J] + jnp.asarray(_OFF_J)[None, :]
    len16 = jnp.broadcast_to(
        jnp.maximum(Xi_mle.astype(jnp.float32), 1.0)[:, :, None],
        (B, NMUL, EMB))
    len16 = jnp.asarray(len16)

    inv = 1.0 / np.sqrt(1.0 + BN_EPS)
    s1 = bn1_g * inv
    w1f = Wl1[jnp.asarray(_WL1_ROW_PERM), :] * s1[None, :]
    c1 = (bl1 * s1 + bn1_b).reshape(1, D1)
    s2 = bn2_g * inv
    w2f = Wl2 * s2[None, :]
    c2 = (bl2 * s2 + bn2_b).reshape(1, D2)

    a_out, b_out, fo_out = _sc_gather(w2one, w2mul, w1cat, idx_all, len16)
    a2 = a_out.reshape(B, PAIRS * EMB)
    b2 = b_out.reshape(B, PAIRS * EMB)
    return _tc_mlp(a2, b2, fo_out, w1f, c1, w2f, c2, bias)


# speedup vs baseline: 4.4077x; 1.0053x over previous
import functools

import numpy as np
import jax
import jax.numpy as jnp
from jax import lax
from jax.experimental import pallas as pl
from jax.experimental.pallas import tpu as pltpu
from jax.experimental.pallas import tpu_sc as plsc

F = 26
NONE = 21
NMUL = 5
V1 = 100000
VM = 100001
EMB = 16
MAXN = 20
B = 4096
PAIRS = F * (F - 1) // 2
D1 = 128
D2 = 128
BN_EPS = 1e-5

NW = 32
BPW = B // NW


_my_pairs = []
for _i in range(NONE):
    for _j in range(_i + 1, NONE):
        _my_pairs.append((_i, _j))
for _i in range(NONE):
    for _m in range(NMUL):
        _my_pairs.append((_i, NONE + _m))
_t10 = {}
for _a in range(NMUL):
    for _b in range(_a + 1, NMUL):
        _t10[(_a, _b)] = len(_t10)
        _my_pairs.append((NONE + _a, NONE + _b))
assert len(_my_pairs) == PAIRS

_slot_view = np.zeros(2 * PAIRS, np.int64)
_slot_field = np.zeros(2 * PAIRS, np.int64)
for _p, (_i, _j) in enumerate(_my_pairs):
    _slot_view[_p], _slot_field[_p] = _i, _j
    _slot_view[PAIRS + _p], _slot_field[PAIRS + _p] = _j, _i

_cov = np.zeros((F, F), np.int64)
for _s in range(2 * PAIRS):
    _cov[_slot_view[_s], _slot_field[_s]] += 1
assert (np.diag(_cov) == 0).all() and ((_cov + np.eye(F, dtype=np.int64)) == 1).all()

_iu, _ju = np.triu_indices(F, 1)
_ref_q = {(int(a), int(b)): q for q, (a, b) in enumerate(zip(_iu, _ju))}
_pair_perm = np.array([_ref_q[p] for p in _my_pairs], np.int64)
_WL1_ROW_PERM = (_pair_perm[:, None] * EMB + np.arange(EMB)[None, :]).reshape(-1)

_groups = [(v, m) for v in range(NONE) for m in range(NMUL)]
for _a in range(NMUL):
    for _b in range(NMUL):
        if _a != _b:
            _groups.append((NONE + _a, _b))
assert len(_groups) == 125


def _mul_dest(v, m):
    j = NONE + m
    if v < j:
        if v < NONE:
            return 210 + v * NMUL + m
        return 315 + _t10[(v - NONE, m)]
    return PAIRS + 315 + _t10[(m, v - NONE)]


_GROUP_DEST = np.array([_mul_dest(v, m) for v, m in _groups], np.int64)
for _g in range(105):
    assert _GROUP_DEST[_g] == 210 + _g
_MM_GROUPS = [(g, _groups[g][1], int(_GROUP_DEST[g])) for g in range(105, 125)]
_sv = set(int(s) for s in _GROUP_DEST)
_mul_slots = set(int(s) for s in range(2 * PAIRS) if _slot_field[s] >= NONE)
assert _sv == _mul_slots

AB0, MR0, FO0, IDXL = 0, 656, 3160, 3360
_SRC = np.full(IDXL, 121, np.int64)
_OFF = np.zeros(IDXL, np.int64)
for _s in range(2 * PAIRS):
    if _slot_field[_s] < NONE:
        _SRC[_s] = _slot_field[_s]
        _OFF[_s] = _slot_view[_s] * V1
for _g, (_v, _m) in enumerate(_groups):
    for _k in range(MAXN):
        _e = MR0 + _g * MAXN + _k
        _SRC[_e] = NONE + _m * MAXN + _k
        _OFF[_e] = _v * VM
for _k in range(32):
    _e = FO0 + _k
    if _k < NONE:
        _SRC[_e] = _k
        _OFF[_e] = 0
    else:
        _OFF[_e] = V1
for _m in range(NMUL):
    for _k in range(32):
        _e = FO0 + 32 + _m * 32 + _k
        if _k < MAXN:
            _SRC[_e] = NONE + _m * MAXN + _k
        _OFF[_e] = V1
_SRC_J = _SRC.astype(np.int32)
_OFF_J = _OFF.astype(np.int32)

IDXW = IDXL


def _sc_gather(w2one, w2mul, w1cat, idx_all, rinv):
    mesh = plsc.VectorSubcoreMesh(core_axis_name="c", subcore_axis_name="s")

    @functools.partial(
        pl.kernel,
        out_type=[
            jax.ShapeDtypeStruct((B, PAIRS, EMB), jnp.float32),
            jax.ShapeDtypeStruct((B, PAIRS, EMB), jnp.float32),
            jax.ShapeDtypeStruct((B, EMB), jnp.float32),
        ],
        mesh=mesh,
        scratch_types=[
            pltpu.VMEM((IDXW,), jnp.int32),
            pltpu.VMEM((IDXW,), jnp.int32),
            pltpu.VMEM((2 * PAIRS, EMB), jnp.float32),
            pltpu.VMEM((2 * PAIRS, EMB), jnp.float32),
            pltpu.VMEM((2500, EMB), jnp.float32),
            pltpu.VMEM((2500, EMB), jnp.float32),
            pltpu.VMEM((192, EMB), jnp.float32),
            pltpu.VMEM((192, EMB), jnp.float32),
            pltpu.VMEM((BPW, EMB), jnp.float32),
            pltpu.VMEM((BPW, NMUL * EMB), jnp.float32),
            pltpu.SemaphoreType.DMA,
            pltpu.SemaphoreType.DMA,
            pltpu.SemaphoreType.DMA,
            pltpu.SemaphoreType.DMA,
            pltpu.SemaphoreType.DMA,
            pltpu.SemaphoreType.DMA,
            pltpu.SemaphoreType.DMA,
            pltpu.SemaphoreType.DMA,
        ],
        compiler_params=pltpu.CompilerParams(use_tc_tiling_on_sc=False),
    )
    def k(w2one_r, w2mul_r, w1cat_r, idx_r, rinv_r, a_out, b_out, fo_out,
          idx_v0, idx_v1, ab_v0, ab_v1, mr_v0, mr_v1, fo_v0, fo_v1, fov_v,
          rinv_v, isem0, isem1, gab0, gab1, gmr0, gmr1, gfo0, gfo1):
        idx_v = (idx_v0, idx_v1)
        ab_v = (ab_v0, ab_v1)
        mr_v = (mr_v0, mr_v1)
        fo_v = (fo_v0, fo_v1)
        isem = (isem0, isem1)
        gab = (gab0, gab1)
        gmr = (gmr0, gmr1)
        gfo = (gfo0, gfo1)

        wid = lax.axis_index("c") * 16 + lax.axis_index("s")
        base = wid * BPW

        def idx_desc(buf, row):
            return pltpu.make_async_copy(idx_r.at[row], idx_v[buf], isem[buf])

        def gather_descs(buf):
            return (
                pltpu.make_async_copy(
                    w2one_r.at[idx_v[buf].at[pl.ds(AB0, 2 * PAIRS)]],
                    ab_v[buf], gab[buf]),
                pltpu.make_async_copy(
                    w2mul_r.at[idx_v[buf].at[pl.ds(MR0, 2500)]],
                    mr_v[buf], gmr[buf]),
                pltpu.make_async_copy(
                    w1cat_r.at[idx_v[buf].at[pl.ds(FO0, 192)]],
                    fo_v[buf], gfo[buf]),
            )

        def fire_gathers(buf):
            for d in gather_descs(buf):
                d.start()

        def wait_gathers(buf):
            for d in gather_descs(buf):
                d.wait()

        def reduce_row(buf, e):
            @pl.loop(0, 105)
            def _grp(g):
                r0 = g * MAXN
                acc = mr_v[buf][pl.ds(r0, 1), :]
                for kk in range(1, MAXN):
                    acc = acc + mr_v[buf][pl.ds(r0 + kk, 1), :]
                m = lax.rem(g, NMUL)
                rv = rinv_v[pl.ds(e, 1), pl.ds(m * EMB, EMB)]
                ab_v[buf][pl.ds(210 + g, 1), :] = acc * rv

            for g, m, dest in _MM_GROUPS:
                r0 = g * MAXN
                acc = mr_v[buf][pl.ds(r0, 1), :]
                for kk in range(1, MAXN):
                    acc = acc + mr_v[buf][pl.ds(r0 + kk, 1), :]
                rv = rinv_v[pl.ds(e, 1), pl.ds(m * EMB, EMB)]
                ab_v[buf][pl.ds(dest, 1), :] = acc * rv

            facc = fo_v[buf][pl.ds(0, 1), :]
            for kk in range(1, 32):
                facc = facc + fo_v[buf][pl.ds(kk, 1), :]
            for m in range(NMUL):
                mac = fo_v[buf][pl.ds(32 + m * 32, 1), :]
                for kk in range(1, 32):
                    mac = mac + fo_v[buf][pl.ds(32 + m * 32 + kk, 1), :]
                rv = rinv_v[pl.ds(e, 1), pl.ds(m * EMB, EMB)]
                facc = facc + mac * rv
            fov_v[pl.ds(e, 1), :] = facc

        pltpu.sync_copy(rinv_r.at[pl.ds(base, BPW)], rinv_v)
        pltpu.sync_copy(idx_r.at[base], idx_v[0])
        fire_gathers(0)
        idx_desc(1, base + 1).start()

        @pl.loop(0, BPW // 2)
        def _pair(g):
            for h in (0, 1):
                e = 2 * g + h
                bi = base + e
                buf = h
                wait_gathers(buf)
                if h == 0:
                    idx_desc(1 - buf, base + e + 1).wait()
                    fire_gathers(1 - buf)
                else:
                    idx_desc(1 - buf, base + lax.min(e + 1, BPW - 1)).wait()

                    @pl.when(e + 1 < BPW)
                    def _():
                        fire_gathers(1 - buf)

                reduce_row(buf, e)
                idx_desc(buf, base + lax.min(e + 2, BPW - 1)).start()
                pltpu.sync_copy(ab_v[buf].at[pl.ds(0, PAIRS)], a_out.at[bi])
                pltpu.sync_copy(ab_v[buf].at[pl.ds(PAIRS, PAIRS)],
                                b_out.at[bi])

        idx_desc(1, base).wait()
        pltpu.sync_copy(fov_v, fo_out.at[pl.ds(base, BPW)])

    return k(w2one, w2mul, w1cat, idx_all, rinv)


def _tc_body(a_ref, b_ref, fo_ref, w1_ref, c1_ref, w2_ref, c2_ref, bias_ref,
             o_ref):
    d = a_ref[...] * b_ref[...]
    x1 = jnp.dot(d, w1_ref[...], preferred_element_type=jnp.float32)
    x1 = jnp.maximum(x1 + c1_ref[...], 0.0)
    x2 = jnp.dot(x1, w2_ref[...], preferred_element_type=jnp.float32)
    x2 = jnp.maximum(x2 + c2_ref[...], 0.0)
    tot = bias_ref[0] + fo_ref[...].sum(axis=1) + d.sum(axis=1) + x2.sum(axis=1)
    o_ref[...] = tot


def _tc_mlp(a2, b2, fov, w1f, c1, w2f, c2, bias):
    blk = 128
    nblk = B // blk
    out = pl.pallas_call(
        _tc_body,
        grid=(nblk,),
        in_specs=[
            pl.BlockSpec((blk, PAIRS * EMB), lambda i: (i, 0)),
            pl.BlockSpec((blk, PAIRS * EMB), lambda i: (i, 0)),
            pl.BlockSpec((blk, EMB), lambda i: (i, 0)),
            pl.BlockSpec((PAIRS * EMB, D1), lambda i: (0, 0)),
            pl.BlockSpec((1, D1), lambda i: (0, 0)),
            pl.BlockSpec((D1, D2), lambda i: (0, 0)),
            pl.BlockSpec((1, D2), lambda i: (0, 0)),
            pl.BlockSpec(memory_space=pltpu.SMEM),
        ],
        out_specs=pl.BlockSpec((blk,), lambda i: (i,)),
        out_shape=jax.ShapeDtypeStruct((B,), jnp.float32),
    )(a2, b2, fov, w1f, c1, w2f, c2, bias)
    return out


def kernel(Xi_one, Xi_mul, Xi_mle, bias, W1_one, W1_mul, W2_one, W2_mul,
           Wl1, bl1, Wl2, bl2, bn1_g, bn1_b, bn2_g, bn2_b):
    Xi_one = Xi_one.astype(jnp.int32)
    Xi_mul = Xi_mul.astype(jnp.int32)
    Xi_mle = Xi_mle.astype(jnp.int32)

    w2one = W2_one.reshape(F * V1, EMB)
    w2mul = W2_mul.reshape(F * VM, EMB)
    w1flat = jnp.concatenate([W1_one[:, 0], W1_mul[:, 0]])
    w1cat = jnp.pad(w1flat[:, None], ((0, 0), (0, EMB - 1)))

    xcat = jnp.concatenate(
        [Xi_one, Xi_mul.reshape(B, NMUL * MAXN), jnp.zeros((B, 1), jnp.int32)],
        axis=1)
    idx_all = xcat[:, _SRC_J] + jnp.asarray(_OFF_J)[None, :]
    rinv16 = jnp.broadcast_to(
        (1.0 / jnp.maximum(Xi_mle.astype(jnp.float32), 1.0))[:, :, None],
        (B, NMUL, EMB)).reshape(B, NMUL * EMB)

    inv = 1.0 / np.sqrt(1.0 + BN_EPS)
    s1 = bn1_g * inv
    w1f = Wl1[jnp.asarray(_WL1_ROW_PERM), :] * s1[None, :]
    c1 = (bl1 * s1 + bn1_b).reshape(1, D1)
    s2 = bn2_g * inv
    w2f = Wl2 * s2[None, :]
    c2 = (bl2 * s2 + bn2_b).reshape(1, D2)

    a_out, b_out, fo_out = _sc_gather(w2one, w2mul, w1cat, idx_all, rinv16)
    a2 = a_out.reshape(B, PAIRS * EMB)
    b2 = b_out.reshape(B, PAIRS * EMB)
    return _tc_mlp(a2, b2, fo_out, w1f, c1, w2f, c2, bias)


# speedup vs baseline: 15.1931x; 3.4470x over previous
import functools

import numpy as np
import jax
import jax.numpy as jnp
from jax import lax
from jax.experimental import pallas as pl
from jax.experimental.pallas import tpu as pltpu
from jax.experimental.pallas import tpu_sc as plsc

F = 26
NONE = 21
NMUL = 5
V1 = 100000
VM = 100001
EMB = 16
MAXN = 20
B = 4096
PAIRS = F * (F - 1) // 2
D1 = 128
D2 = 128
BN_EPS = 1e-5

NW = 32
BPW = B // NW

ROWW = F * EMB + EMB
W1C = F * EMB
NCH = ROWW // EMB

ONE0, MUL0, IDXW = 0, 24, 128

_my_pairs = []
for _i in range(NONE):
    for _j in range(_i + 1, NONE):
        _my_pairs.append((_i, _j))
for _i in range(NONE):
    for _m in range(NMUL):
        _my_pairs.append((_i, NONE + _m))
for _a in range(NMUL):
    for _b in range(_a + 1, NMUL):
        _my_pairs.append((NONE + _a, NONE + _b))
assert len(_my_pairs) == PAIRS

_iu, _ju = np.triu_indices(F, 1)
_ref_q = {(int(a), int(b)): q for q, (a, b) in enumerate(zip(_iu, _ju))}
_pair_perm = np.array([_ref_q[p] for p in _my_pairs], np.int64)
_WL1_ROW_PERM = (_pair_perm[:, None] * EMB + np.arange(EMB)[None, :]).reshape(-1)


def _sc_gather(w2oneT, w2mulT, idx_all, rinv):
    mesh = plsc.VectorSubcoreMesh(core_axis_name="c", subcore_axis_name="s")

    @functools.partial(
        pl.kernel,
        out_type=[
            jax.ShapeDtypeStruct((B, PAIRS, EMB), jnp.float32),
            jax.ShapeDtypeStruct((B, EMB), jnp.float32),
        ],
        mesh=mesh,
        scratch_types=[
            pltpu.VMEM((IDXW,), jnp.int32),
            pltpu.VMEM((IDXW,), jnp.int32),
            pltpu.VMEM((NMUL, EMB), jnp.float32),
            pltpu.VMEM((NMUL, EMB), jnp.float32),
            pltpu.VMEM((NONE, ROWW), jnp.float32),
            pltpu.VMEM((NONE, ROWW), jnp.float32),
            pltpu.VMEM((MAXN * NMUL, ROWW), jnp.float32),
            pltpu.VMEM((MAXN * NMUL, ROWW), jnp.float32),
            pltpu.VMEM((NMUL * NCH, EMB), jnp.float32),
            pltpu.VMEM((PAIRS, EMB), jnp.float32),
            pltpu.VMEM((PAIRS, EMB), jnp.float32),
            pltpu.VMEM((BPW, EMB), jnp.float32),
            pltpu.SemaphoreType.DMA,
            pltpu.SemaphoreType.DMA,
            pltpu.SemaphoreType.DMA,
            pltpu.SemaphoreType.DMA,
            pltpu.SemaphoreType.DMA,
            pltpu.SemaphoreType.DMA,
            pltpu.SemaphoreType.DMA,
            pltpu.SemaphoreType.DMA,
        ],
        compiler_params=pltpu.CompilerParams(use_tc_tiling_on_sc=False),
    )
    def k(w2oneT_r, w2mulT_r, idx_r, rinv_r, w_out, fo_out,
          idx_v0, idx_v1, rinv_v0, rinv_v1, ot_v0, ot_v1, mt_v0, mt_v1,
          macc_v, wij_v0, wij_v1, fov_v,
          isem0, isem1, got0, got1, gmt0, gmt1, osem0, osem1):
        idx_v = (idx_v0, idx_v1)
        rinv_v = (rinv_v0, rinv_v1)
        ot_v = (ot_v0, ot_v1)
        mt_v = (mt_v0, mt_v1)
        wij_v = (wij_v0, wij_v1)
        isem = (isem0, isem1)
        got = (got0, got1)
        gmt = (gmt0, gmt1)
        osem = (osem0, osem1)

        wid = lax.axis_index("c") * 16 + lax.axis_index("s")
        base = wid * BPW

        def in_descs(buf, row):
            return (
                pltpu.make_async_copy(idx_r.at[row], idx_v[buf], isem[buf]),
                pltpu.make_async_copy(rinv_r.at[row], rinv_v[buf], isem[buf]),
            )

        def gather_descs(buf):
            return (
                pltpu.make_async_copy(
                    w2mulT_r.at[idx_v[buf].at[pl.ds(MUL0, NMUL * MAXN)]],
                    mt_v[buf], gmt[buf]),
                pltpu.make_async_copy(
                    w2oneT_r.at[idx_v[buf].at[pl.ds(ONE0, NONE)]],
                    ot_v[buf], got[buf]),
            )

        def out_desc(buf, row):
            return pltpu.make_async_copy(wij_v[buf], w_out.at[row], osem[buf])

        def reduce_row(buf, e):
            @pl.loop(0, NCH)
            def _chunk(j):
                c0 = j * EMB
                for m in range(NMUL):
                    r0 = m * MAXN
                    acc = mt_v[buf][pl.ds(r0, 1), pl.ds(c0, EMB)]
                    for kk in range(1, MAXN):
                        acc = acc + mt_v[buf][pl.ds(r0 + kk, 1),
                                              pl.ds(c0, EMB)]
                    macc_v[pl.ds(m * NCH + j, 1), :] = (
                        acc * rinv_v[buf][pl.ds(m, 1), :])

            for p, (i, j) in enumerate(_my_pairs):
                if j < NONE:
                    a = ot_v[buf][pl.ds(j, 1), pl.ds(i * EMB, EMB)]
                    bb = ot_v[buf][pl.ds(i, 1), pl.ds(j * EMB, EMB)]
                elif i < NONE:
                    m = j - NONE
                    a = macc_v[pl.ds(m * NCH + i, 1), :]
                    bb = ot_v[buf][pl.ds(i, 1), pl.ds(j * EMB, EMB)]
                else:
                    ma, mb = i - NONE, j - NONE
                    a = macc_v[pl.ds(mb * NCH + i, 1), :]
                    bb = macc_v[pl.ds(ma * NCH + j, 1), :]
                wij_v[buf][pl.ds(p, 1), :] = a * bb

            facc = ot_v[buf][pl.ds(0, 1), pl.ds(W1C, EMB)]
            for f in range(1, NONE):
                facc = facc + ot_v[buf][pl.ds(f, 1), pl.ds(W1C, EMB)]
            for m in range(NMUL):
                facc = facc + macc_v[pl.ds(m * NCH + NCH - 1, 1), :]
            fov_v[pl.ds(e, 1), :] = facc

        for d in in_descs(0, base):
            d.start()
        for d in in_descs(0, base):
            d.wait()
        for d in gather_descs(0):
            d.start()
        for d in in_descs(1, base + 1):
            d.start()

        @pl.loop(0, BPW // 2)
        def _pair_loop(g):
            for h in (0, 1):
                e = 2 * g + h
                bi = base + e
                buf = h
                if h == 0:
                    for d in in_descs(1 - buf, base + e + 1):
                        d.wait()

                    @pl.when(e > 0)
                    def _():
                        out_desc(1 - buf, bi - 1).wait()

                    for d in gather_descs(1 - buf):
                        d.start()
                else:
                    for d in in_descs(1 - buf,
                                      base + lax.min(e + 1, BPW - 1)):
                        d.wait()
                    out_desc(1 - buf, bi - 1).wait()

                    @pl.when(e + 1 < BPW)
                    def _():
                        for d in gather_descs(1 - buf):
                            d.start()

                descs = gather_descs(buf)
                descs[0].wait()
                descs[1].wait()
                pre = base + lax.min(e + 2, BPW - 1)
                in_descs(buf, pre)[0].start()
                reduce_row(buf, e)
                in_descs(buf, pre)[1].start()
                out_desc(buf, bi).start()

        for d in in_descs(1, base):
            d.wait()
        out_desc(1, base).wait()
        pltpu.sync_copy(fov_v, fo_out.at[pl.ds(base, BPW)])

    return k(w2oneT, w2mulT, idx_all, rinv)


def _tc_body(d_ref, fo_ref, w1_ref, c1_ref, w2_ref, c2_ref, bias_ref, o_ref):
    d = d_ref[...]
    x1 = jnp.dot(d, w1_ref[...], preferred_element_type=jnp.float32)
    x1 = jnp.maximum(x1 + c1_ref[...], 0.0)
    x2 = jnp.dot(x1, w2_ref[...], preferred_element_type=jnp.float32)
    x2 = jnp.maximum(x2 + c2_ref[...], 0.0)
    tot = bias_ref[0] + fo_ref[...].sum(axis=1) + d.sum(axis=1) + x2.sum(axis=1)
    o_ref[...] = tot


def _tc_mlp(d2, fov, w1f, c1, w2f, c2, bias):
    blk = 128
    nblk = B // blk
    out = pl.pallas_call(
        _tc_body,
        grid=(nblk,),
        in_specs=[
            pl.BlockSpec((blk, PAIRS * EMB), lambda i: (i, 0)),
            pl.BlockSpec((blk, EMB), lambda i: (i, 0)),
            pl.BlockSpec((PAIRS * EMB, D1), lambda i: (0, 0)),
            pl.BlockSpec((1, D1), lambda i: (0, 0)),
            pl.BlockSpec((D1, D2), lambda i: (0, 0)),
            pl.BlockSpec((1, D2), lambda i: (0, 0)),
            pl.BlockSpec(memory_space=pltpu.SMEM),
        ],
        out_specs=pl.BlockSpec((blk,), lambda i: (i,)),
        out_shape=jax.ShapeDtypeStruct((B,), jnp.float32),
    )(d2, fov, w1f, c1, w2f, c2, bias)
    return out


def kernel(Xi_one, Xi_mul, Xi_mle, bias, W1_one, W1_mul, W2_one, W2_mul,
           Wl1, bl1, Wl2, bl2, bn1_g, bn1_b, bn2_g, bn2_b):
    Xi_one = Xi_one.astype(jnp.int32)
    Xi_mul = Xi_mul.astype(jnp.int32)
    Xi_mle = Xi_mle.astype(jnp.int32)

    w2oneT = jnp.concatenate(
        [jnp.transpose(W2_one, (1, 0, 2)).reshape(V1, F * EMB), W1_one,
         jnp.zeros((V1, EMB - 1), jnp.float32)], axis=1)
    w2mulT = jnp.concatenate(
        [jnp.transpose(W2_mul, (1, 0, 2)).reshape(VM, F * EMB), W1_mul,
         jnp.zeros((VM, EMB - 1), jnp.float32)], axis=1)

    idx_all = jnp.concatenate(
        [Xi_one, jnp.zeros((B, MUL0 - NONE), jnp.int32),
         Xi_mul.reshape(B, NMUL * MAXN),
         jnp.zeros((B, IDXW - MUL0 - NMUL * MAXN), jnp.int32)], axis=1)
    rinv16 = jnp.broadcast_to(
        (1.0 / jnp.maximum(Xi_mle.astype(jnp.float32), 1.0))[:, :, None],
        (B, NMUL, EMB))
    rinv16 = jnp.asarray(rinv16)

    inv = 1.0 / np.sqrt(1.0 + BN_EPS)
    s1 = bn1_g * inv
    w1f = Wl1[jnp.asarray(_WL1_ROW_PERM), :] * s1[None, :]
    c1 = (bl1 * s1 + bn1_b).reshape(1, D1)
    s2 = bn2_g * inv
    w2f = Wl2 * s2[None, :]
    c2 = (bl2 * s2 + bn2_b).reshape(1, D2)

    w_out, fo_out = _sc_gather(w2oneT, w2mulT, idx_all, rinv16)
    d2 = w_out.reshape(B, PAIRS * EMB)
    return _tc_mlp(d2, fo_out, w1f, c1, w2f, c2, bias)
